# Initial kernel scaffold; baseline (speedup 1.0000x reference)
import jax
import jax.numpy as jnp
from jax.experimental import pallas as pl


def kernel(x, edge_index, W1, b1, W2, b2, Wf1, bf1, Wf2, bf2):
    raise NotImplementedError("write your pallas kernel here")


# speedup vs baseline: 33.2223x; 33.2223x over previous
import functools

import jax
import jax.numpy as jnp
from jax import lax
from jax.experimental import pallas as pl
from jax.experimental.pallas import tpu as pltpu
from jax.experimental.pallas import tpu_sc as plsc

N = 10000
F = 16
NC = 2
NS = 16
NW = NC * NS
CHUNK = 128
NPAD = 10240
ROWS_PT = NPAD // NS

_mesh = plsc.VectorSubcoreMesh(core_axis_name="c", subcore_axis_name="s")


@functools.lru_cache(maxsize=None)
def _make_deg(K):
  @functools.partial(
      pl.kernel,
      out_type=jax.ShapeDtypeStruct((NC, NPAD, F), jnp.float32),
      mesh=_mesh,
      compiler_params=pltpu.CompilerParams(use_tc_tiling_on_sc=False),
      scratch_types=[
          pltpu.VMEM((K, CHUNK), jnp.int32),
          pltpu.VMEM((CHUNK, F), jnp.float32),
          pltpu.VMEM_SHARED((NPAD, F), jnp.float32),
          pltpu.SemaphoreType.DMA,
      ],
  )
  def deg(dst_hbm, zrows_hbm, out_hbm, di_v, ones_v, acc_sh, sem):
    c = lax.axis_index("c")
    s = lax.axis_index("s")
    cp = pltpu.async_copy(dst_hbm.at[c, s], di_v, sem)

    def setone(j, carry):
      ones_v[j] = jnp.ones((F,), jnp.float32)
      return carry

    lax.fori_loop(0, CHUNK, setone, 0)
    pltpu.sync_copy(zrows_hbm, acc_sh.at[pl.ds(s * ROWS_PT, ROWS_PT)])
    cp.wait()
    plsc.subcore_barrier()

    def body(j, carry):
      pltpu.sync_copy(ones_v, acc_sh.at[di_v.at[j]], add=True)
      return carry

    lax.fori_loop(0, K, body, 0)
    plsc.subcore_barrier()
    pltpu.sync_copy(acc_sh.at[pl.ds(s * ROWS_PT, ROWS_PT)],
                    out_hbm.at[c, pl.ds(s * ROWS_PT, ROWS_PT)])

  return deg


@functools.lru_cache(maxsize=None)
def _make_agg(K):
  @functools.partial(
      pl.kernel,
      out_type=jax.ShapeDtypeStruct((NC, NPAD, F), jnp.float32),
      mesh=_mesh,
      compiler_params=pltpu.CompilerParams(use_tc_tiling_on_sc=False),
      scratch_types=[
          pltpu.VMEM((K, CHUNK), jnp.int32),
          pltpu.VMEM((K, CHUNK), jnp.int32),
          pltpu.VMEM((CHUNK, F), jnp.float32),
          pltpu.VMEM_SHARED((NPAD, F), jnp.float32),
          pltpu.SemaphoreType.DMA,
          pltpu.SemaphoreType.DMA,
      ],
  )
  def agg(g_hbm, src_hbm, dst_hbm, zrows_hbm, out_hbm,
          si_v, di_v, rows_v, acc_sh, sem1, sem2):
    c = lax.axis_index("c")
    s = lax.axis_index("s")
    cp1 = pltpu.async_copy(src_hbm.at[c, s], si_v, sem1)
    cp2 = pltpu.async_copy(dst_hbm.at[c, s], di_v, sem2)
    pltpu.sync_copy(zrows_hbm, acc_sh.at[pl.ds(s * ROWS_PT, ROWS_PT)])
    cp1.wait()
    cp2.wait()
    plsc.subcore_barrier()

    def body(j, carry):
      pltpu.sync_copy(g_hbm.at[si_v.at[j]], rows_v)
      pltpu.sync_copy(rows_v, acc_sh.at[di_v.at[j]], add=True)
      return carry

    lax.fori_loop(0, K, body, 0)
    plsc.subcore_barrier()
    pltpu.sync_copy(acc_sh.at[pl.ds(s * ROWS_PT, ROWS_PT)],
                    out_hbm.at[c, pl.ds(s * ROWS_PT, ROWS_PT)])

  return agg


def _tc1(x, W1, degp):
  def body(x_ref, w1_ref, degp_ref, g1_ref, dinv_ref):
    deg = degp_ref[0, :, 0:1] + degp_ref[1, :, 0:1] + 1.0
    dinv = lax.rsqrt(deg)
    dinv_ref[...] = dinv
    h1 = jnp.dot(x_ref[...], w1_ref[...], preferred_element_type=jnp.float32)
    g1_ref[:N] = h1 * dinv[:N]
    g1_ref[N:] = jnp.zeros((NPAD - N, F), jnp.float32)

  return pl.pallas_call(
      body,
      out_shape=(jax.ShapeDtypeStruct((NPAD, F), jnp.float32),
                 jax.ShapeDtypeStruct((NPAD, 1), jnp.float32)),
  )(x, W1, degp)


def _tc2(accp, g1, dinv, b1, W2):
  def body(accp_ref, g1_ref, dinv_ref, b1_ref, w2_ref, g2_ref):
    acc = accp_ref[0, :N] + accp_ref[1, :N] + g1_ref[:N]
    h = jnp.maximum(dinv_ref[:N] * acc + b1_ref[...], 0.0)
    g2 = jnp.dot(h, w2_ref[...], preferred_element_type=jnp.float32) * dinv_ref[:N]
    g2_ref[:N] = g2
    g2_ref[N:] = jnp.zeros((NPAD - N, F), jnp.float32)

  return pl.pallas_call(
      body,
      out_shape=jax.ShapeDtypeStruct((NPAD, F), jnp.float32),
  )(accp, g1, dinv, b1, W2)


def _tc3(accp, g2, dinv, b2, Wf1, bf1, Wf2, bf2):
  def body(accp_ref, g2_ref, dinv_ref, b2_ref, wf1_ref, bf1_ref,
           wf2_ref, bf2_ref, out_ref):
    acc = accp_ref[0, :N] + accp_ref[1, :N] + g2_ref[:N]
    h = jnp.maximum(dinv_ref[:N] * acc + b2_ref[...], 0.0)
    m = jnp.maximum(
        jnp.dot(h, wf1_ref[...], preferred_element_type=jnp.float32)
        + bf1_ref[...], 0.0)
    out_ref[...] = (jnp.dot(m, wf2_ref[...], preferred_element_type=jnp.float32)
                    + bf2_ref[...])

  return pl.pallas_call(
      body,
      out_shape=jax.ShapeDtypeStruct((N, 1), jnp.float32),
  )(accp, g2, dinv, b2, Wf1, bf1, Wf2, bf2)


def kernel(x, edge_index, W1, b1, W2, b2, Wf1, bf1, Wf2, bf2):
  src = edge_index[0].astype(jnp.int32)
  dst = edge_index[1].astype(jnp.int32)
  E = src.shape[0]
  ept = -(-E // NW)
  K = -(-ept // CHUNK)
  pad = NW * K * CHUNK - E
  srcp = jnp.concatenate([src, jnp.full((pad,), N, jnp.int32)]).reshape(NC, NS, K, CHUNK)
  dstp = jnp.concatenate([dst, jnp.full((pad,), N, jnp.int32)]).reshape(NC, NS, K, CHUNK)
  zrows = jnp.zeros((ROWS_PT, F), jnp.float32)

  degp = _make_deg(K)(dstp, zrows)
  g1, dinv = _tc1(x, W1, degp)
  acc1 = _make_agg(K)(g1, srcp, dstp, zrows)
  g2 = _tc2(acc1, g1, dinv, b1.reshape(1, F), W2)
  acc2 = _make_agg(K)(g2, srcp, dstp, zrows)
  return _tc3(acc2, g2, dinv, b2.reshape(1, F), Wf1,
              bf1.reshape(1, 8), Wf2, bf2.reshape(1, 1))


# speedup vs baseline: 38.7504x; 1.1664x over previous
import functools

import jax
import jax.numpy as jnp
from jax import lax
from jax.experimental import pallas as pl
from jax.experimental.pallas import tpu as pltpu
from jax.experimental.pallas import tpu_sc as plsc

N = 10000
F = 16
NC = 2
NS = 16
NW = NC * NS
CHUNK = 128
NPAD = 10240
ROWS_PT = NPAD // NS
NBUF = 8

_mesh = plsc.VectorSubcoreMesh(core_axis_name="c", subcore_axis_name="s")


@functools.lru_cache(maxsize=None)
def _make_deg(K):
  @functools.partial(
      pl.kernel,
      out_type=jax.ShapeDtypeStruct((NC, NPAD, F), jnp.float32),
      mesh=_mesh,
      compiler_params=pltpu.CompilerParams(use_tc_tiling_on_sc=False),
      scratch_types=[
          pltpu.VMEM((K, CHUNK), jnp.int32),
          pltpu.VMEM((CHUNK, F), jnp.float32),
          pltpu.VMEM_SHARED((NPAD, F), jnp.float32),
          pltpu.SemaphoreType.DMA,
      ],
  )
  def deg(dst_hbm, zrows_hbm, out_hbm, di_v, ones_v, acc_sh, sem):
    c = lax.axis_index("c")
    s = lax.axis_index("s")
    cp = pltpu.async_copy(dst_hbm.at[c, s], di_v, sem)

    def setone(j, carry):
      ones_v[j] = jnp.ones((F,), jnp.float32)
      return carry

    lax.fori_loop(0, CHUNK, setone, 0)
    pltpu.sync_copy(zrows_hbm, acc_sh.at[pl.ds(s * ROWS_PT, ROWS_PT)])
    cp.wait()
    plsc.subcore_barrier()

    GRP = 16
    assert K % GRP == 0

    def body(g, carry):
      base = g * GRP
      for i in range(GRP):
        pltpu.async_copy(ones_v, acc_sh.at[di_v.at[base + i]], sem, add=True)
      for i in range(GRP):
        pltpu.make_async_copy(ones_v, acc_sh.at[di_v.at[base + i]],
                              sem).wait()
      return carry

    lax.fori_loop(0, K // GRP, body, 0)
    plsc.subcore_barrier()
    pltpu.sync_copy(acc_sh.at[pl.ds(s * ROWS_PT, ROWS_PT)],
                    out_hbm.at[c, pl.ds(s * ROWS_PT, ROWS_PT)])

  return deg


@functools.lru_cache(maxsize=None)
def _make_agg(K):
  @functools.partial(
      pl.kernel,
      out_type=jax.ShapeDtypeStruct((NC, NPAD, F), jnp.float32),
      mesh=_mesh,
      compiler_params=pltpu.CompilerParams(use_tc_tiling_on_sc=False),
      scratch_types=[
          pltpu.VMEM((K, CHUNK), jnp.int32),
          pltpu.VMEM((K, CHUNK), jnp.int32),
          pltpu.VMEM((NBUF, CHUNK, F), jnp.float32),
          pltpu.VMEM_SHARED((NPAD, F), jnp.float32),
          pltpu.SemaphoreType.DMA,
          pltpu.SemaphoreType.DMA,
          pltpu.SemaphoreType.DMA((NBUF,)),
          pltpu.SemaphoreType.DMA((NBUF,)),
      ],
  )
  def agg(g_hbm, src_hbm, dst_hbm, zrows_hbm, out_hbm,
          si_v, di_v, rows_v, acc_sh, sem1, sem2, gsem, ssem):
    c = lax.axis_index("c")
    s = lax.axis_index("s")
    cp1 = pltpu.async_copy(src_hbm.at[c, s], si_v, sem1)
    cp2 = pltpu.async_copy(dst_hbm.at[c, s], di_v, sem2)
    pltpu.sync_copy(zrows_hbm, acc_sh.at[pl.ds(s * ROWS_PT, ROWS_PT)])
    cp1.wait()
    cp2.wait()
    plsc.subcore_barrier()

    def g_start(j, b):
      pltpu.async_copy(g_hbm.at[si_v.at[j]], rows_v.at[b], gsem.at[b])

    def g_wait(j, b):
      pltpu.make_async_copy(g_hbm.at[si_v.at[j]], rows_v.at[b],
                            gsem.at[b]).wait()

    def s_start(j, b):
      pltpu.async_copy(rows_v.at[b], acc_sh.at[di_v.at[j]], ssem.at[b],
                       add=True)

    def s_wait(j, b):
      pltpu.make_async_copy(rows_v.at[b], acc_sh.at[di_v.at[j]],
                            ssem.at[b]).wait()

    assert K % NBUF == 0
    NG = K // NBUF
    for b in range(NBUF):
      g_start(b, b)

    def outer(g, carry):
      base = g * NBUF
      for b in range(NBUF):
        g_wait(base + b, b)
        s_start(base + b, b)
      for b in range(NBUF):
        s_wait(base + b, b)

        @pl.when(g + 1 < NG)
        def _():
          g_start(base + NBUF + b, b)

      return carry

    lax.fori_loop(0, NG, outer, 0)
    plsc.subcore_barrier()
    pltpu.sync_copy(acc_sh.at[pl.ds(s * ROWS_PT, ROWS_PT)],
                    out_hbm.at[c, pl.ds(s * ROWS_PT, ROWS_PT)])

  return agg


def _tc1(x, W1, degp):
  def body(x_ref, w1_ref, degp_ref, g1_ref, dinv_ref):
    deg = degp_ref[0, :, 0:1] + degp_ref[1, :, 0:1] + 1.0
    dinv = lax.rsqrt(deg)
    dinv_ref[...] = dinv
    h1 = jnp.dot(x_ref[...], w1_ref[...], preferred_element_type=jnp.float32)
    g1_ref[:N] = h1 * dinv[:N]
    g1_ref[N:] = jnp.zeros((NPAD - N, F), jnp.float32)

  return pl.pallas_call(
      body,
      out_shape=(jax.ShapeDtypeStruct((NPAD, F), jnp.float32),
                 jax.ShapeDtypeStruct((NPAD, 1), jnp.float32)),
  )(x, W1, degp)


def _tc2(accp, g1, dinv, b1, W2):
  def body(accp_ref, g1_ref, dinv_ref, b1_ref, w2_ref, g2_ref):
    acc = accp_ref[0, :N] + accp_ref[1, :N] + g1_ref[:N]
    h = jnp.maximum(dinv_ref[:N] * acc + b1_ref[...], 0.0)
    g2 = jnp.dot(h, w2_ref[...], preferred_element_type=jnp.float32) * dinv_ref[:N]
    g2_ref[:N] = g2
    g2_ref[N:] = jnp.zeros((NPAD - N, F), jnp.float32)

  return pl.pallas_call(
      body,
      out_shape=jax.ShapeDtypeStruct((NPAD, F), jnp.float32),
  )(accp, g1, dinv, b1, W2)


def _tc3(accp, g2, dinv, b2, Wf1, bf1, Wf2, bf2):
  def body(accp_ref, g2_ref, dinv_ref, b2_ref, wf1_ref, bf1_ref,
           wf2_ref, bf2_ref, out_ref):
    acc = accp_ref[0, :N] + accp_ref[1, :N] + g2_ref[:N]
    h = jnp.maximum(dinv_ref[:N] * acc + b2_ref[...], 0.0)
    m = jnp.maximum(
        jnp.dot(h, wf1_ref[...], preferred_element_type=jnp.float32)
        + bf1_ref[...], 0.0)
    out_ref[...] = (jnp.dot(m, wf2_ref[...], preferred_element_type=jnp.float32)
                    + bf2_ref[...])

  return pl.pallas_call(
      body,
      out_shape=jax.ShapeDtypeStruct((N, 1), jnp.float32),
  )(accp, g2, dinv, b2, Wf1, bf1, Wf2, bf2)


def kernel(x, edge_index, W1, b1, W2, b2, Wf1, bf1, Wf2, bf2):
  src = edge_index[0].astype(jnp.int32)
  dst = edge_index[1].astype(jnp.int32)
  E = src.shape[0]
  ept = -(-E // NW)
  K = -(-ept // CHUNK)
  K = -(-K // 16) * 16
  pad = NW * K * CHUNK - E
  srcp = jnp.concatenate([src, jnp.full((pad,), N, jnp.int32)]).reshape(NC, NS, K, CHUNK)
  dstp = jnp.concatenate([dst, jnp.full((pad,), N, jnp.int32)]).reshape(NC, NS, K, CHUNK)
  zrows = jnp.zeros((ROWS_PT, F), jnp.float32)

  degp = _make_deg(K)(dstp, zrows)
  g1, dinv = _tc1(x, W1, degp)
  acc1 = _make_agg(K)(g1, srcp, dstp, zrows)
  g2 = _tc2(acc1, g1, dinv, b1.reshape(1, F), W2)
  acc2 = _make_agg(K)(g2, srcp, dstp, zrows)
  return _tc3(acc2, g2, dinv, b2.reshape(1, F), Wf1,
              bf1.reshape(1, 8), Wf2, bf2.reshape(1, 1))


# speedup vs baseline: 54.4777x; 1.4059x over previous
import functools

import jax
import jax.numpy as jnp
from jax import lax
from jax.experimental import pallas as pl
from jax.experimental.pallas import tpu as pltpu
from jax.experimental.pallas import tpu_sc as plsc

N = 10000
F = 16
NC = 2
NS = 16
NW = NC * NS
CHUNK = 128
NPAD = 10240
ROWS_PT = NPAD // NS
NBUF = 8

_mesh = plsc.VectorSubcoreMesh(core_axis_name="c", subcore_axis_name="s")


@functools.lru_cache(maxsize=None)
def _make_deg(K):
  @functools.partial(
      pl.kernel,
      out_type=jax.ShapeDtypeStruct((NC, NPAD, F), jnp.float32),
      mesh=_mesh,
      compiler_params=pltpu.CompilerParams(use_tc_tiling_on_sc=False),
      scratch_types=[
          pltpu.VMEM((K, CHUNK), jnp.int32),
          pltpu.VMEM((CHUNK, F), jnp.float32),
          pltpu.VMEM_SHARED((NPAD, F), jnp.float32),
          pltpu.SemaphoreType.DMA,
      ],
  )
  def deg(dst_hbm, zrows_hbm, out_hbm, di_v, ones_v, acc_sh, sem):
    c = lax.axis_index("c")
    s = lax.axis_index("s")
    cp = pltpu.async_copy(dst_hbm.at[c, s], di_v, sem)

    def setone(j, carry):
      ones_v[j] = jnp.ones((F,), jnp.float32)
      return carry

    lax.fori_loop(0, CHUNK, setone, 0)
    pltpu.sync_copy(zrows_hbm, acc_sh.at[pl.ds(s * ROWS_PT, ROWS_PT)])
    cp.wait()
    plsc.subcore_barrier()

    GRP = 40
    assert K % GRP == 0

    def body(g, carry):
      base = g * GRP
      for i in range(GRP):
        pltpu.async_copy(ones_v, acc_sh.at[di_v.at[base + i]], sem, add=True)
      for i in range(GRP):
        pltpu.make_async_copy(ones_v, acc_sh.at[di_v.at[base + i]],
                              sem).wait()
      return carry

    lax.fori_loop(0, K // GRP, body, 0)
    plsc.subcore_barrier()
    pltpu.sync_copy(acc_sh.at[pl.ds(s * ROWS_PT, ROWS_PT)],
                    out_hbm.at[c, pl.ds(s * ROWS_PT, ROWS_PT)])

  return deg


@functools.lru_cache(maxsize=None)
def _make_agg(K):
  @functools.partial(
      pl.kernel,
      out_type=jax.ShapeDtypeStruct((NC, NPAD, F), jnp.float32),
      mesh=_mesh,
      compiler_params=pltpu.CompilerParams(use_tc_tiling_on_sc=False),
      scratch_types=[
          pltpu.VMEM((K, CHUNK), jnp.int32),
          pltpu.VMEM((K, CHUNK), jnp.int32),
          pltpu.VMEM((NBUF, CHUNK, F), jnp.float32),
          pltpu.VMEM_SHARED((NPAD, F), jnp.float32),
          pltpu.VMEM_SHARED((NPAD, F), jnp.float32),
          pltpu.SemaphoreType.DMA,
          pltpu.SemaphoreType.DMA,
          pltpu.SemaphoreType.DMA((NBUF,)),
          pltpu.SemaphoreType.DMA((NBUF,)),
      ],
  )
  def agg(g_hbm, src_hbm, dst_hbm, zrows_hbm, out_hbm,
          si_v, di_v, rows_v, acc_sh, g_sh, sem1, sem2, gsem, ssem):
    c = lax.axis_index("c")
    s = lax.axis_index("s")
    cp1 = pltpu.async_copy(src_hbm.at[c, s], si_v, sem1)
    cp2 = pltpu.async_copy(dst_hbm.at[c, s], di_v, sem2)
    pltpu.sync_copy(g_hbm.at[pl.ds(s * ROWS_PT, ROWS_PT)],
                    g_sh.at[pl.ds(s * ROWS_PT, ROWS_PT)])
    pltpu.sync_copy(zrows_hbm, acc_sh.at[pl.ds(s * ROWS_PT, ROWS_PT)])
    cp1.wait()
    cp2.wait()
    plsc.subcore_barrier()

    def g_start(j, b):
      pltpu.async_copy(g_sh.at[si_v.at[j]], rows_v.at[b], gsem.at[b])

    def g_wait(j, b):
      pltpu.make_async_copy(g_sh.at[si_v.at[j]], rows_v.at[b],
                            gsem.at[b]).wait()

    def s_start(j, b):
      pltpu.async_copy(rows_v.at[b], acc_sh.at[di_v.at[j]], ssem.at[b],
                       add=True)

    def s_wait(j, b):
      pltpu.make_async_copy(rows_v.at[b], acc_sh.at[di_v.at[j]],
                            ssem.at[b]).wait()

    assert K % NBUF == 0
    NG = K // NBUF
    for b in range(NBUF):
      g_start(b, b)

    def outer(g, carry):
      base = g * NBUF
      for b in range(NBUF):
        g_wait(base + b, b)
        s_start(base + b, b)
      for b in range(NBUF):
        s_wait(base + b, b)

        @pl.when(g + 1 < NG)
        def _():
          g_start(base + NBUF + b, b)

      return carry

    lax.fori_loop(0, NG, outer, 0)
    plsc.subcore_barrier()
    pltpu.sync_copy(acc_sh.at[pl.ds(s * ROWS_PT, ROWS_PT)],
                    out_hbm.at[c, pl.ds(s * ROWS_PT, ROWS_PT)])

  return agg


def _tc1(x, W1, degp):
  def body(x_ref, w1_ref, degp_ref, g1_ref, dinv_ref):
    deg = degp_ref[0, :, 0:1] + degp_ref[1, :, 0:1] + 1.0
    dinv = lax.rsqrt(deg)
    dinv_ref[...] = dinv
    h1 = jnp.dot(x_ref[...], w1_ref[...], preferred_element_type=jnp.float32)
    g1_ref[:N] = h1 * dinv[:N]
    g1_ref[N:] = jnp.zeros((NPAD - N, F), jnp.float32)

  return pl.pallas_call(
      body,
      out_shape=(jax.ShapeDtypeStruct((NPAD, F), jnp.float32),
                 jax.ShapeDtypeStruct((NPAD, 1), jnp.float32)),
  )(x, W1, degp)


def _tc2(accp, g1, dinv, b1, W2):
  def body(accp_ref, g1_ref, dinv_ref, b1_ref, w2_ref, g2_ref):
    acc = accp_ref[0, :N] + accp_ref[1, :N] + g1_ref[:N]
    h = jnp.maximum(dinv_ref[:N] * acc + b1_ref[...], 0.0)
    g2 = jnp.dot(h, w2_ref[...], preferred_element_type=jnp.float32) * dinv_ref[:N]
    g2_ref[:N] = g2
    g2_ref[N:] = jnp.zeros((NPAD - N, F), jnp.float32)

  return pl.pallas_call(
      body,
      out_shape=jax.ShapeDtypeStruct((NPAD, F), jnp.float32),
  )(accp, g1, dinv, b1, W2)


def _tc3(accp, g2, dinv, b2, Wf1, bf1, Wf2, bf2):
  def body(accp_ref, g2_ref, dinv_ref, b2_ref, wf1_ref, bf1_ref,
           wf2_ref, bf2_ref, out_ref):
    acc = accp_ref[0, :N] + accp_ref[1, :N] + g2_ref[:N]
    h = jnp.maximum(dinv_ref[:N] * acc + b2_ref[...], 0.0)
    m = jnp.maximum(
        jnp.dot(h, wf1_ref[...], preferred_element_type=jnp.float32)
        + bf1_ref[...], 0.0)
    out_ref[...] = (jnp.dot(m, wf2_ref[...], preferred_element_type=jnp.float32)
                    + bf2_ref[...])

  return pl.pallas_call(
      body,
      out_shape=jax.ShapeDtypeStruct((N, 1), jnp.float32),
  )(accp, g2, dinv, b2, Wf1, bf1, Wf2, bf2)


def kernel(x, edge_index, W1, b1, W2, b2, Wf1, bf1, Wf2, bf2):
  src = edge_index[0].astype(jnp.int32)
  dst = edge_index[1].astype(jnp.int32)
  E = src.shape[0]
  ept = -(-E // NW)
  K = -(-ept // CHUNK)
  K = -(-K // 16) * 16
  pad = NW * K * CHUNK - E
  srcp = jnp.concatenate([src, jnp.full((pad,), N, jnp.int32)]).reshape(NC, NS, K, CHUNK)
  dstp = jnp.concatenate([dst, jnp.full((pad,), N, jnp.int32)]).reshape(NC, NS, K, CHUNK)
  zrows = jnp.zeros((ROWS_PT, F), jnp.float32)

  degp = _make_deg(K)(dstp, zrows)
  g1, dinv = _tc1(x, W1, degp)
  acc1 = _make_agg(K)(g1, srcp, dstp, zrows)
  g2 = _tc2(acc1, g1, dinv, b1.reshape(1, F), W2)
  acc2 = _make_agg(K)(g2, srcp, dstp, zrows)
  return _tc3(acc2, g2, dinv, b2.reshape(1, F), Wf1,
              bf1.reshape(1, 8), Wf2, bf2.reshape(1, 1))


# speedup vs baseline: 90.4745x; 1.6608x over previous
import functools

import jax
import jax.numpy as jnp
from jax import lax
from jax.experimental import pallas as pl
from jax.experimental.pallas import tpu as pltpu
from jax.experimental.pallas import tpu_sc as plsc

N = 10000
F = 16
NC = 2
NS = 16
NW = NC * NS
CHUNK = 128
NPAD = 10240
ROWS_PT = NPAD // NS
NBUF = 8
PK = NPAD // 8
PKN = N // 8

_mesh = plsc.VectorSubcoreMesh(core_axis_name="c", subcore_axis_name="s")


def _worker_span(c, s, n_chunks):
  wid = c * NS + s
  per, extra = n_chunks // NW, n_chunks % NW
  base = wid * per + jnp.minimum(wid, extra)
  cnt = jnp.where(wid < extra, per + 1, per)
  return base, cnt


@functools.lru_cache(maxsize=None)
def _make_deg(n_chunks):
  cpt = n_chunks // NW + (1 if n_chunks % NW else 0)

  @functools.partial(
      pl.kernel,
      out_type=jax.ShapeDtypeStruct((NC, NPAD, F), jnp.float32),
      mesh=_mesh,
      compiler_params=pltpu.CompilerParams(use_tc_tiling_on_sc=False),
      scratch_types=[
          pltpu.VMEM((cpt, CHUNK), jnp.int32),
          pltpu.VMEM((CHUNK, F), jnp.float32),
          pltpu.VMEM_SHARED((NPAD, F), jnp.float32),
          pltpu.SemaphoreType.DMA,
          pltpu.SemaphoreType.DMA,
      ],
  )
  def deg(ei_hbm, zrows_hbm, out_hbm, di_v, ones_v, acc_sh, sem, isem):
    c = lax.axis_index("c")
    s = lax.axis_index("s")
    base, cnt = _worker_span(c, s, n_chunks)
    per = n_chunks // NW
    cp = pltpu.async_copy(ei_hbm.at[1, pl.ds(base, per)],
                          di_v.at[pl.ds(0, per)], isem)

    def setone(j, carry):
      ones_v[j] = jnp.ones((F,), jnp.float32)
      return carry

    lax.fori_loop(0, CHUNK, setone, 0)
    pltpu.sync_copy(zrows_hbm, acc_sh.at[pl.ds(s * ROWS_PT, ROWS_PT)])
    cp.wait()

    @pl.when(cnt > per)
    def _():
      pltpu.sync_copy(ei_hbm.at[1, pl.ds(base + per, 1)],
                      di_v.at[pl.ds(per, 1)])

    plsc.subcore_barrier()

    GRP = 40
    for lo in range(0, cpt, GRP):
      hi = min(lo + GRP, cpt)
      for j in range(lo, hi):
        @pl.when(j < cnt)
        def _():
          pltpu.async_copy(ones_v, acc_sh.at[di_v.at[j]], sem, add=True)
      for j in range(lo, hi):
        @pl.when(j < cnt)
        def _():
          pltpu.make_async_copy(ones_v, acc_sh.at[di_v.at[j]], sem).wait()

    plsc.subcore_barrier()
    pltpu.sync_copy(acc_sh.at[pl.ds(s * ROWS_PT, ROWS_PT)],
                    out_hbm.at[c, pl.ds(s * ROWS_PT, ROWS_PT)])

  return deg


@functools.lru_cache(maxsize=None)
def _make_agg(n_chunks):
  cpt = n_chunks // NW + (1 if n_chunks % NW else 0)
  ng = -(-cpt // NBUF)

  @functools.partial(
      pl.kernel,
      out_type=jax.ShapeDtypeStruct((NC, NPAD, F), jnp.float32),
      mesh=_mesh,
      compiler_params=pltpu.CompilerParams(use_tc_tiling_on_sc=False),
      scratch_types=[
          pltpu.VMEM((cpt, CHUNK), jnp.int32),
          pltpu.VMEM((cpt, CHUNK), jnp.int32),
          pltpu.VMEM((NBUF, CHUNK, F), jnp.float32),
          pltpu.VMEM_SHARED((NPAD, F), jnp.float32),
          pltpu.VMEM_SHARED((NPAD, F), jnp.float32),
          pltpu.SemaphoreType.DMA,
          pltpu.SemaphoreType.DMA,
          pltpu.SemaphoreType.DMA((NBUF,)),
          pltpu.SemaphoreType.DMA((NBUF,)),
      ],
  )
  def agg(g_hbm, ei_hbm, zrows_hbm, out_hbm,
          si_v, di_v, rows_v, acc_sh, g_sh, sem1, sem2, gsem, ssem):
    c = lax.axis_index("c")
    s = lax.axis_index("s")
    base, cnt = _worker_span(c, s, n_chunks)
    per = n_chunks // NW
    cp1 = pltpu.async_copy(ei_hbm.at[0, pl.ds(base, per)],
                           si_v.at[pl.ds(0, per)], sem1)
    cp2 = pltpu.async_copy(ei_hbm.at[1, pl.ds(base, per)],
                           di_v.at[pl.ds(0, per)], sem2)
    pltpu.sync_copy(g_hbm.at[pl.ds(s * ROWS_PT, ROWS_PT)],
                    g_sh.at[pl.ds(s * ROWS_PT, ROWS_PT)])
    pltpu.sync_copy(zrows_hbm, acc_sh.at[pl.ds(s * ROWS_PT, ROWS_PT)])
    cp1.wait()
    cp2.wait()

    @pl.when(cnt > per)
    def _():
      pltpu.sync_copy(ei_hbm.at[0, pl.ds(base + per, 1)],
                      si_v.at[pl.ds(per, 1)])
      pltpu.sync_copy(ei_hbm.at[1, pl.ds(base + per, 1)],
                      di_v.at[pl.ds(per, 1)])

    plsc.subcore_barrier()

    def g_start(j, b):
      pltpu.async_copy(g_sh.at[si_v.at[j]], rows_v.at[b], gsem.at[b])

    def g_wait(j, b):
      pltpu.make_async_copy(g_sh.at[si_v.at[j]], rows_v.at[b],
                            gsem.at[b]).wait()

    def s_start(j, b):
      pltpu.async_copy(rows_v.at[b], acc_sh.at[di_v.at[j]], ssem.at[b],
                       add=True)

    def s_wait(j, b):
      pltpu.make_async_copy(rows_v.at[b], acc_sh.at[di_v.at[j]],
                            ssem.at[b]).wait()

    for b in range(NBUF):
      g_start(b, b)

    def outer(g, carry):
      gbase = g * NBUF
      for b in range(NBUF):
        j = gbase + b

        @pl.when(j < cnt)
        def _():
          g_wait(j, b)
          s_start(j, b)

      for b in range(NBUF):
        j = gbase + b
        j2 = j + NBUF

        @pl.when(j < cnt)
        def _():
          s_wait(j, b)

        @pl.when(j2 < cnt)
        def _():
          g_start(j2, b)

      return carry

    lax.fori_loop(0, ng, outer, 0)
    plsc.subcore_barrier()
    pltpu.sync_copy(acc_sh.at[pl.ds(s * ROWS_PT, ROWS_PT)],
                    out_hbm.at[c, pl.ds(s * ROWS_PT, ROWS_PT)])

  return agg


def _tc1(x3, W1, degp):
  def body(x_ref, w1_ref, degp_ref, g1_ref, dinv_ref):
    dinv = lax.rsqrt(degp_ref[0] + degp_ref[1] + 1.0)
    dinv_ref[...] = dinv
    for a in range(8):
      ha = jnp.dot(x_ref[:, a, :], w1_ref[...],
                   preferred_element_type=jnp.float32)
      g1_ref[:PKN, 16 * a:16 * (a + 1)] = ha * dinv[:PKN, 16 * a:16 * (a + 1)]
    g1_ref[PKN:, :] = jnp.zeros((PK - PKN, 128), jnp.float32)

  return pl.pallas_call(
      body,
      out_shape=(jax.ShapeDtypeStruct((PK, 128), jnp.float32),
                 jax.ShapeDtypeStruct((PK, 128), jnp.float32)),
  )(x3, W1, degp)


def _tc2(accp, g1, dinv, b1t, W2):
  def body(accp_ref, g1_ref, dinv_ref, b1t_ref, w2_ref, g2_ref):
    acc = accp_ref[0] + accp_ref[1] + g1_ref[...]
    h = jnp.maximum(dinv_ref[...] * acc + b1t_ref[...], 0.0)
    for a in range(8):
      ha = h[:, 16 * a:16 * (a + 1)]
      g2a = jnp.dot(ha, w2_ref[...], preferred_element_type=jnp.float32)
      g2_ref[:, 16 * a:16 * (a + 1)] = g2a * dinv_ref[:, 16 * a:16 * (a + 1)]

  return pl.pallas_call(
      body,
      out_shape=jax.ShapeDtypeStruct((PK, 128), jnp.float32),
  )(accp, g1, dinv, b1t, W2)


def _tc3(accp, g2, dinv, b2t, Wf1, bf1, Wf2, bf2):
  def body(accp_ref, g2_ref, dinv_ref, b2t_ref, wf1_ref, bf1_ref,
           wf2_ref, bf2_ref, out_ref):
    acc = accp_ref[0] + accp_ref[1] + g2_ref[...]
    h = jnp.maximum(dinv_ref[...] * acc + b2t_ref[...], 0.0)
    cols = []
    for a in range(8):
      ha = h[:, 16 * a:16 * (a + 1)]
      m = jnp.maximum(
          jnp.dot(ha, wf1_ref[...], preferred_element_type=jnp.float32)
          + bf1_ref[...], 0.0)
      cols.append(jnp.dot(m, wf2_ref[...], preferred_element_type=jnp.float32)
                  + bf2_ref[...])
    out_ref[...] = jnp.concatenate(cols, axis=1)

  return pl.pallas_call(
      body,
      out_shape=jax.ShapeDtypeStruct((PK, 8), jnp.float32),
  )(accp, g2, dinv, b2t, Wf1, bf1, Wf2, bf2)


def kernel(x, edge_index, W1, b1, W2, b2, Wf1, bf1, Wf2, bf2):
  E = edge_index.shape[1]
  n_chunks = E // CHUNK
  ei3 = edge_index.astype(jnp.int32).reshape(2, n_chunks, CHUNK)
  x3 = x.reshape(PKN, 8, 128)
  zrows = jnp.zeros((ROWS_PT, F), jnp.float32)
  b1t = jnp.tile(b1, 8).reshape(1, 128)
  b2t = jnp.tile(b2, 8).reshape(1, 128)

  degp = _make_deg(n_chunks)(ei3, zrows)
  degp_pk = degp.reshape(NC, PK, 128)
  g1, dinv = _tc1(x3, W1, degp_pk)
  acc1 = _make_agg(n_chunks)(g1.reshape(NPAD, F), ei3, zrows)
  g2 = _tc2(acc1.reshape(NC, PK, 128), g1, dinv, b1t, W2)
  acc2 = _make_agg(n_chunks)(g2.reshape(NPAD, F), ei3, zrows)
  out_pk = _tc3(acc2.reshape(NC, PK, 128), g2, dinv, b2t, Wf1,
                bf1.reshape(1, 8), Wf2, bf2.reshape(1, 1))
  return out_pk.reshape(NPAD, 1)[:N]


# speedup vs baseline: 92.7606x; 1.0253x over previous
import functools

import jax
import jax.numpy as jnp
from jax import lax
from jax.experimental import pallas as pl
from jax.experimental.pallas import tpu as pltpu
from jax.experimental.pallas import tpu_sc as plsc

N = 10000
F = 16
NC = 2
NS = 16
NW = NC * NS
CHUNK = 256
NPAD = 10240
ROWS_PT = NPAD // NS
NBUF = 8
PK = NPAD // 8
PKN = N // 8

_mesh = plsc.VectorSubcoreMesh(core_axis_name="c", subcore_axis_name="s")


def _worker_span(c, s, n_chunks):
  wid = c * NS + s
  per, extra = n_chunks // NW, n_chunks % NW
  base = wid * per + jnp.minimum(wid, extra)
  cnt = jnp.where(wid < extra, per + 1, per)
  return base, cnt


@functools.lru_cache(maxsize=None)
def _make_deg(n_chunks):
  cpt = n_chunks // NW + (1 if n_chunks % NW else 0)

  @functools.partial(
      pl.kernel,
      out_type=jax.ShapeDtypeStruct((NC, NPAD, F), jnp.float32),
      mesh=_mesh,
      compiler_params=pltpu.CompilerParams(use_tc_tiling_on_sc=False),
      scratch_types=[
          pltpu.VMEM((cpt, CHUNK), jnp.int32),
          pltpu.VMEM((CHUNK, F), jnp.float32),
          pltpu.VMEM_SHARED((NPAD, F), jnp.float32),
          pltpu.SemaphoreType.DMA,
          pltpu.SemaphoreType.DMA,
      ],
  )
  def deg(ei_hbm, zrows_hbm, out_hbm, di_v, ones_v, acc_sh, sem, isem):
    c = lax.axis_index("c")
    s = lax.axis_index("s")
    base, cnt = _worker_span(c, s, n_chunks)
    per = n_chunks // NW
    cp = pltpu.async_copy(ei_hbm.at[1, pl.ds(base, per)],
                          di_v.at[pl.ds(0, per)], isem)

    def setone(j, carry):
      ones_v[j] = jnp.ones((F,), jnp.float32)
      return carry

    lax.fori_loop(0, CHUNK, setone, 0)
    pltpu.sync_copy(zrows_hbm, acc_sh.at[pl.ds(s * ROWS_PT, ROWS_PT)])
    cp.wait()

    @pl.when(cnt > per)
    def _():
      pltpu.sync_copy(ei_hbm.at[1, pl.ds(base + per, 1)],
                      di_v.at[pl.ds(per, 1)])

    plsc.subcore_barrier()

    GRP = 40
    for lo in range(0, cpt, GRP):
      hi = min(lo + GRP, cpt)
      for j in range(lo, hi):
        @pl.when(j < cnt)
        def _():
          pltpu.async_copy(ones_v, acc_sh.at[di_v.at[j]], sem, add=True)
      for j in range(lo, hi):
        @pl.when(j < cnt)
        def _():
          pltpu.make_async_copy(ones_v, acc_sh.at[di_v.at[j]], sem).wait()

    plsc.subcore_barrier()
    pltpu.sync_copy(acc_sh.at[pl.ds(s * ROWS_PT, ROWS_PT)],
                    out_hbm.at[c, pl.ds(s * ROWS_PT, ROWS_PT)])

  return deg


@functools.lru_cache(maxsize=None)
def _make_agg(n_chunks):
  cpt = n_chunks // NW + (1 if n_chunks % NW else 0)
  ng = -(-cpt // NBUF)

  @functools.partial(
      pl.kernel,
      out_type=jax.ShapeDtypeStruct((NC, NPAD, F), jnp.float32),
      mesh=_mesh,
      compiler_params=pltpu.CompilerParams(use_tc_tiling_on_sc=False),
      scratch_types=[
          pltpu.VMEM((cpt, CHUNK), jnp.int32),
          pltpu.VMEM((cpt, CHUNK), jnp.int32),
          pltpu.VMEM((NBUF, CHUNK, F), jnp.float32),
          pltpu.VMEM_SHARED((NPAD, F), jnp.float32),
          pltpu.VMEM_SHARED((NPAD, F), jnp.float32),
          pltpu.SemaphoreType.DMA,
          pltpu.SemaphoreType.DMA,
          pltpu.SemaphoreType.DMA((NBUF,)),
          pltpu.SemaphoreType.DMA((NBUF,)),
      ],
  )
  def agg(g_hbm, ei_hbm, zrows_hbm, out_hbm,
          si_v, di_v, rows_v, acc_sh, g_sh, sem1, sem2, gsem, ssem):
    c = lax.axis_index("c")
    s = lax.axis_index("s")
    base, cnt = _worker_span(c, s, n_chunks)
    per = n_chunks // NW
    cp1 = pltpu.async_copy(ei_hbm.at[0, pl.ds(base, per)],
                           si_v.at[pl.ds(0, per)], sem1)
    cp2 = pltpu.async_copy(ei_hbm.at[1, pl.ds(base, per)],
                           di_v.at[pl.ds(0, per)], sem2)
    pltpu.sync_copy(g_hbm.at[pl.ds(s * ROWS_PT, ROWS_PT)],
                    g_sh.at[pl.ds(s * ROWS_PT, ROWS_PT)])
    pltpu.sync_copy(zrows_hbm, acc_sh.at[pl.ds(s * ROWS_PT, ROWS_PT)])
    cp1.wait()
    cp2.wait()

    @pl.when(cnt > per)
    def _():
      pltpu.sync_copy(ei_hbm.at[0, pl.ds(base + per, 1)],
                      si_v.at[pl.ds(per, 1)])
      pltpu.sync_copy(ei_hbm.at[1, pl.ds(base + per, 1)],
                      di_v.at[pl.ds(per, 1)])

    plsc.subcore_barrier()

    def g_start(j, b):
      pltpu.async_copy(g_sh.at[si_v.at[j]], rows_v.at[b], gsem.at[b])

    def g_wait(j, b):
      pltpu.make_async_copy(g_sh.at[si_v.at[j]], rows_v.at[b],
                            gsem.at[b]).wait()

    def s_start(j, b):
      pltpu.async_copy(rows_v.at[b], acc_sh.at[di_v.at[j]], ssem.at[b],
                       add=True)

    def s_wait(j, b):
      pltpu.make_async_copy(rows_v.at[b], acc_sh.at[di_v.at[j]],
                            ssem.at[b]).wait()

    for b in range(NBUF):
      g_start(b, b)

    def outer(g, carry):
      gbase = g * NBUF
      for b in range(NBUF):
        j = gbase + b

        @pl.when(j < cnt)
        def _():
          g_wait(j, b)
          s_start(j, b)

      for b in range(NBUF):
        j = gbase + b
        j2 = j + NBUF

        @pl.when(j < cnt)
        def _():
          s_wait(j, b)

        @pl.when(j2 < cnt)
        def _():
          g_start(j2, b)

      return carry

    lax.fori_loop(0, ng, outer, 0)
    plsc.subcore_barrier()
    pltpu.sync_copy(acc_sh.at[pl.ds(s * ROWS_PT, ROWS_PT)],
                    out_hbm.at[c, pl.ds(s * ROWS_PT, ROWS_PT)])

  return agg


def _tc1(x3, W1, degp):
  def body(x_ref, w1_ref, degp_ref, g1_ref, dinv_ref):
    dinv = lax.rsqrt(degp_ref[0] + degp_ref[1] + 1.0)
    dinv_ref[...] = dinv
    for a in range(8):
      ha = jnp.dot(x_ref[:, a, :], w1_ref[...],
                   preferred_element_type=jnp.float32)
      g1_ref[:PKN, 16 * a:16 * (a + 1)] = ha * dinv[:PKN, 16 * a:16 * (a + 1)]
    g1_ref[PKN:, :] = jnp.zeros((PK - PKN, 128), jnp.float32)

  return pl.pallas_call(
      body,
      out_shape=(jax.ShapeDtypeStruct((PK, 128), jnp.float32),
                 jax.ShapeDtypeStruct((PK, 128), jnp.float32)),
  )(x3, W1, degp)


def _tc2(accp, g1, dinv, b1t, W2):
  def body(accp_ref, g1_ref, dinv_ref, b1t_ref, w2_ref, g2_ref):
    acc = accp_ref[0] + accp_ref[1] + g1_ref[...]
    h = jnp.maximum(dinv_ref[...] * acc + b1t_ref[...], 0.0)
    for a in range(8):
      ha = h[:, 16 * a:16 * (a + 1)]
      g2a = jnp.dot(ha, w2_ref[...], preferred_element_type=jnp.float32)
      g2_ref[:, 16 * a:16 * (a + 1)] = g2a * dinv_ref[:, 16 * a:16 * (a + 1)]

  return pl.pallas_call(
      body,
      out_shape=jax.ShapeDtypeStruct((PK, 128), jnp.float32),
  )(accp, g1, dinv, b1t, W2)


def _tc3(accp, g2, dinv, b2t, Wf1, bf1, Wf2, bf2):
  def body(accp_ref, g2_ref, dinv_ref, b2t_ref, wf1_ref, bf1_ref,
           wf2_ref, bf2_ref, out_ref):
    acc = accp_ref[0] + accp_ref[1] + g2_ref[...]
    h = jnp.maximum(dinv_ref[...] * acc + b2t_ref[...], 0.0)
    cols = []
    for a in range(8):
      ha = h[:, 16 * a:16 * (a + 1)]
      m = jnp.maximum(
          jnp.dot(ha, wf1_ref[...], preferred_element_type=jnp.float32)
          + bf1_ref[...], 0.0)
      cols.append(jnp.dot(m, wf2_ref[...], preferred_element_type=jnp.float32)
                  + bf2_ref[...])
    out_ref[...] = jnp.concatenate(cols, axis=1)

  return pl.pallas_call(
      body,
      out_shape=jax.ShapeDtypeStruct((PK, 8), jnp.float32),
  )(accp, g2, dinv, b2t, Wf1, bf1, Wf2, bf2)


def kernel(x, edge_index, W1, b1, W2, b2, Wf1, bf1, Wf2, bf2):
  E = edge_index.shape[1]
  n_chunks = E // CHUNK
  ei3 = edge_index.astype(jnp.int32).reshape(2, n_chunks, CHUNK)
  x3 = x.reshape(PKN, 8, 128)
  zrows = jnp.zeros((ROWS_PT, F), jnp.float32)
  b1t = jnp.tile(b1, 8).reshape(1, 128)
  b2t = jnp.tile(b2, 8).reshape(1, 128)

  degp = _make_deg(n_chunks)(ei3, zrows)
  degp_pk = degp.reshape(NC, PK, 128)
  g1, dinv = _tc1(x3, W1, degp_pk)
  acc1 = _make_agg(n_chunks)(g1.reshape(NPAD, F), ei3, zrows)
  g2 = _tc2(acc1.reshape(NC, PK, 128), g1, dinv, b1t, W2)
  acc2 = _make_agg(n_chunks)(g2.reshape(NPAD, F), ei3, zrows)
  out_pk = _tc3(acc2.reshape(NC, PK, 128), g2, dinv, b2t, Wf1,
                bf1.reshape(1, 8), Wf2, bf2.reshape(1, 1))
  return out_pk.reshape(NPAD, 1)[:N]


# speedup vs baseline: 93.4284x; 1.0072x over previous
import functools

import jax
import jax.numpy as jnp
from jax import lax
from jax.experimental import pallas as pl
from jax.experimental.pallas import tpu as pltpu
from jax.experimental.pallas import tpu_sc as plsc

N = 10000
F = 16
NC = 2
NS = 16
NW = NC * NS
CHUNK = 512
NPAD = 10240
ROWS_PT = NPAD // NS
NBUF = 8
PK = NPAD // 8
PKN = N // 8

_mesh = plsc.VectorSubcoreMesh(core_axis_name="c", subcore_axis_name="s")


def _worker_span(c, s, n_chunks):
  wid = c * NS + s
  per, extra = n_chunks // NW, n_chunks % NW
  base = wid * per + jnp.minimum(wid, extra)
  cnt = jnp.where(wid < extra, per + 1, per)
  return base, cnt


@functools.lru_cache(maxsize=None)
def _make_deg(n_chunks):
  cpt = n_chunks // NW + (1 if n_chunks % NW else 0)

  @functools.partial(
      pl.kernel,
      out_type=jax.ShapeDtypeStruct((NC, NPAD, F), jnp.float32),
      mesh=_mesh,
      compiler_params=pltpu.CompilerParams(use_tc_tiling_on_sc=False),
      scratch_types=[
          pltpu.VMEM((cpt, CHUNK), jnp.int32),
          pltpu.VMEM((CHUNK, F), jnp.float32),
          pltpu.VMEM_SHARED((NPAD, F), jnp.float32),
          pltpu.SemaphoreType.DMA,
          pltpu.SemaphoreType.DMA,
      ],
  )
  def deg(ei_hbm, zrows_hbm, out_hbm, di_v, ones_v, acc_sh, sem, isem):
    c = lax.axis_index("c")
    s = lax.axis_index("s")
    base, cnt = _worker_span(c, s, n_chunks)
    per = n_chunks // NW
    cp = pltpu.async_copy(ei_hbm.at[1, pl.ds(base, per)],
                          di_v.at[pl.ds(0, per)], isem)

    def setone(j, carry):
      ones_v[j] = jnp.ones((F,), jnp.float32)
      return carry

    lax.fori_loop(0, CHUNK, setone, 0)
    pltpu.sync_copy(zrows_hbm, acc_sh.at[pl.ds(s * ROWS_PT, ROWS_PT)])
    cp.wait()

    @pl.when(cnt > per)
    def _():
      pltpu.sync_copy(ei_hbm.at[1, pl.ds(base + per, 1)],
                      di_v.at[pl.ds(per, 1)])

    plsc.subcore_barrier()

    GRP = 40
    for lo in range(0, cpt, GRP):
      hi = min(lo + GRP, cpt)
      for j in range(lo, hi):
        @pl.when(j < cnt)
        def _():
          pltpu.async_copy(ones_v, acc_sh.at[di_v.at[j]], sem, add=True)
      for j in range(lo, hi):
        @pl.when(j < cnt)
        def _():
          pltpu.make_async_copy(ones_v, acc_sh.at[di_v.at[j]], sem).wait()

    plsc.subcore_barrier()
    pltpu.sync_copy(acc_sh.at[pl.ds(s * ROWS_PT, ROWS_PT)],
                    out_hbm.at[c, pl.ds(s * ROWS_PT, ROWS_PT)])

  return deg


@functools.lru_cache(maxsize=None)
def _make_agg(n_chunks):
  cpt = n_chunks // NW + (1 if n_chunks % NW else 0)
  ng = -(-cpt // NBUF)

  @functools.partial(
      pl.kernel,
      out_type=jax.ShapeDtypeStruct((NC, NPAD, F), jnp.float32),
      mesh=_mesh,
      compiler_params=pltpu.CompilerParams(use_tc_tiling_on_sc=False),
      scratch_types=[
          pltpu.VMEM((cpt, CHUNK), jnp.int32),
          pltpu.VMEM((cpt, CHUNK), jnp.int32),
          pltpu.VMEM((NBUF, CHUNK, F), jnp.float32),
          pltpu.VMEM_SHARED((NPAD, F), jnp.float32),
          pltpu.VMEM_SHARED((NPAD, F), jnp.float32),
          pltpu.SemaphoreType.DMA,
          pltpu.SemaphoreType.DMA,
          pltpu.SemaphoreType.DMA((NBUF,)),
          pltpu.SemaphoreType.DMA((NBUF,)),
      ],
  )
  def agg(g_hbm, ei_hbm, zrows_hbm, out_hbm,
          si_v, di_v, rows_v, acc_sh, g_sh, sem1, sem2, gsem, ssem):
    c = lax.axis_index("c")
    s = lax.axis_index("s")
    base, cnt = _worker_span(c, s, n_chunks)
    per = n_chunks // NW
    cp1 = pltpu.async_copy(ei_hbm.at[0, pl.ds(base, per)],
                           si_v.at[pl.ds(0, per)], sem1)
    cp2 = pltpu.async_copy(ei_hbm.at[1, pl.ds(base, per)],
                           di_v.at[pl.ds(0, per)], sem2)
    pltpu.sync_copy(g_hbm.at[pl.ds(s * ROWS_PT, ROWS_PT)],
                    g_sh.at[pl.ds(s * ROWS_PT, ROWS_PT)])
    pltpu.sync_copy(zrows_hbm, acc_sh.at[pl.ds(s * ROWS_PT, ROWS_PT)])
    cp1.wait()
    cp2.wait()

    @pl.when(cnt > per)
    def _():
      pltpu.sync_copy(ei_hbm.at[0, pl.ds(base + per, 1)],
                      si_v.at[pl.ds(per, 1)])
      pltpu.sync_copy(ei_hbm.at[1, pl.ds(base + per, 1)],
                      di_v.at[pl.ds(per, 1)])

    plsc.subcore_barrier()

    def g_start(j, b):
      pltpu.async_copy(g_sh.at[si_v.at[j]], rows_v.at[b], gsem.at[b])

    def g_wait(j, b):
      pltpu.make_async_copy(g_sh.at[si_v.at[j]], rows_v.at[b],
                            gsem.at[b]).wait()

    def s_start(j, b):
      pltpu.async_copy(rows_v.at[b], acc_sh.at[di_v.at[j]], ssem.at[b],
                       add=True)

    def s_wait(j, b):
      pltpu.make_async_copy(rows_v.at[b], acc_sh.at[di_v.at[j]],
                            ssem.at[b]).wait()

    for b in range(NBUF):
      g_start(b, b)

    def outer(g, carry):
      gbase = g * NBUF
      for b in range(NBUF):
        j = gbase + b

        @pl.when(j < cnt)
        def _():
          g_wait(j, b)
          s_start(j, b)

      for b in range(NBUF):
        j = gbase + b
        j2 = j + NBUF

        @pl.when(j < cnt)
        def _():
          s_wait(j, b)

        @pl.when(j2 < cnt)
        def _():
          g_start(j2, b)

      return carry

    lax.fori_loop(0, ng, outer, 0)
    plsc.subcore_barrier()
    pltpu.sync_copy(acc_sh.at[pl.ds(s * ROWS_PT, ROWS_PT)],
                    out_hbm.at[c, pl.ds(s * ROWS_PT, ROWS_PT)])

  return agg


def _tc1(x3, W1, degp):
  def body(x_ref, w1_ref, degp_ref, g1_ref, dinv_ref):
    dinv = lax.rsqrt(degp_ref[0] + degp_ref[1] + 1.0)
    dinv_ref[...] = dinv
    for a in range(8):
      ha = jnp.dot(x_ref[:, a, :], w1_ref[...],
                   preferred_element_type=jnp.float32)
      g1_ref[:PKN, 16 * a:16 * (a + 1)] = ha * dinv[:PKN, 16 * a:16 * (a + 1)]
    g1_ref[PKN:, :] = jnp.zeros((PK - PKN, 128), jnp.float32)

  return pl.pallas_call(
      body,
      out_shape=(jax.ShapeDtypeStruct((PK, 128), jnp.float32),
                 jax.ShapeDtypeStruct((PK, 128), jnp.float32)),
  )(x3, W1, degp)


def _tc2(accp, g1, dinv, b1t, W2):
  def body(accp_ref, g1_ref, dinv_ref, b1t_ref, w2_ref, g2_ref):
    acc = accp_ref[0] + accp_ref[1] + g1_ref[...]
    h = jnp.maximum(dinv_ref[...] * acc + b1t_ref[...], 0.0)
    for a in range(8):
      ha = h[:, 16 * a:16 * (a + 1)]
      g2a = jnp.dot(ha, w2_ref[...], preferred_element_type=jnp.float32)
      g2_ref[:, 16 * a:16 * (a + 1)] = g2a * dinv_ref[:, 16 * a:16 * (a + 1)]

  return pl.pallas_call(
      body,
      out_shape=jax.ShapeDtypeStruct((PK, 128), jnp.float32),
  )(accp, g1, dinv, b1t, W2)


def _tc3(accp, g2, dinv, b2t, Wf1, bf1, Wf2, bf2):
  def body(accp_ref, g2_ref, dinv_ref, b2t_ref, wf1_ref, bf1_ref,
           wf2_ref, bf2_ref, out_ref):
    acc = accp_ref[0] + accp_ref[1] + g2_ref[...]
    h = jnp.maximum(dinv_ref[...] * acc + b2t_ref[...], 0.0)
    cols = []
    for a in range(8):
      ha = h[:, 16 * a:16 * (a + 1)]
      m = jnp.maximum(
          jnp.dot(ha, wf1_ref[...], preferred_element_type=jnp.float32)
          + bf1_ref[...], 0.0)
      cols.append(jnp.dot(m, wf2_ref[...], preferred_element_type=jnp.float32)
                  + bf2_ref[...])
    out_ref[...] = jnp.concatenate(cols, axis=1)

  return pl.pallas_call(
      body,
      out_shape=jax.ShapeDtypeStruct((PK, 8), jnp.float32),
  )(accp, g2, dinv, b2t, Wf1, bf1, Wf2, bf2)


def kernel(x, edge_index, W1, b1, W2, b2, Wf1, bf1, Wf2, bf2):
  E = edge_index.shape[1]
  n_chunks = E // CHUNK
  ei3 = edge_index.astype(jnp.int32).reshape(2, n_chunks, CHUNK)
  x3 = x.reshape(PKN, 8, 128)
  zrows = jnp.zeros((ROWS_PT, F), jnp.float32)
  b1t = jnp.tile(b1, 8).reshape(1, 128)
  b2t = jnp.tile(b2, 8).reshape(1, 128)

  degp = _make_deg(n_chunks)(ei3, zrows)
  degp_pk = degp.reshape(NC, PK, 128)
  g1, dinv = _tc1(x3, W1, degp_pk)
  acc1 = _make_agg(n_chunks)(g1.reshape(NPAD, F), ei3, zrows)
  g2 = _tc2(acc1.reshape(NC, PK, 128), g1, dinv, b1t, W2)
  acc2 = _make_agg(n_chunks)(g2.reshape(NPAD, F), ei3, zrows)
  out_pk = _tc3(acc2.reshape(NC, PK, 128), g2, dinv, b2t, Wf1,
                bf1.reshape(1, 8), Wf2, bf2.reshape(1, 1))
  return out_pk.reshape(NPAD, 1)[:N]


# speedup vs baseline: 97.8397x; 1.0472x over previous
import functools

import jax
import jax.numpy as jnp
from jax import lax
from jax.experimental import pallas as pl
from jax.experimental.pallas import tpu as pltpu
from jax.experimental.pallas import tpu_sc as plsc

N = 10000
F = 16
NC = 2
NS = 16
NW = NC * NS
CHUNK = 512
NPAD = 10240
ROWS_PT = NPAD // NS
NBUF = 8
PK = NPAD // 8
PKN = N // 8

_mesh = plsc.VectorSubcoreMesh(core_axis_name="c", subcore_axis_name="s")


def _worker_span(c, s, n_chunks):
  wid = (NC - 1 - c) * NS + s
  per, extra = n_chunks // NW, n_chunks % NW
  base = wid * per + jnp.minimum(wid, extra)
  cnt = jnp.where(wid < extra, per + 1, per)
  return base, cnt


@functools.lru_cache(maxsize=None)
def _make_deg(n_chunks):
  cpt = n_chunks // NW + (1 if n_chunks % NW else 0)

  @functools.partial(
      pl.kernel,
      out_type=jax.ShapeDtypeStruct((NC, NPAD, F), jnp.float32),
      mesh=_mesh,
      compiler_params=pltpu.CompilerParams(use_tc_tiling_on_sc=False),
      scratch_types=[
          pltpu.VMEM((cpt, CHUNK), jnp.int32),
          pltpu.VMEM((CHUNK, F), jnp.float32),
          pltpu.VMEM_SHARED((NPAD, F), jnp.float32),
          pltpu.SemaphoreType.DMA,
          pltpu.SemaphoreType.DMA,
      ],
  )
  def deg(ei_hbm, zrows_hbm, ones_hbm, out_hbm, di_v, ones_v, acc_sh,
          sem, isem):
    c = lax.axis_index("c")
    s = lax.axis_index("s")
    base, cnt = _worker_span(c, s, n_chunks)
    per = n_chunks // NW
    cp = pltpu.async_copy(ei_hbm.at[1, pl.ds(base, per)],
                          di_v.at[pl.ds(0, per)], isem)
    pltpu.sync_copy(ones_hbm, ones_v)
    pltpu.sync_copy(zrows_hbm, acc_sh.at[pl.ds(s * ROWS_PT, ROWS_PT)])
    cp.wait()

    @pl.when(cnt > per)
    def _():
      pltpu.sync_copy(ei_hbm.at[1, pl.ds(base + per, 1)],
                      di_v.at[pl.ds(per, 1)])

    plsc.subcore_barrier()

    GRP = 40
    for lo in range(0, cpt, GRP):
      hi = min(lo + GRP, cpt)
      for j in range(lo, hi):
        @pl.when(j < cnt)
        def _():
          pltpu.async_copy(ones_v, acc_sh.at[di_v.at[j]], sem, add=True)
      for j in range(lo, hi):
        @pl.when(j < cnt)
        def _():
          pltpu.make_async_copy(ones_v, acc_sh.at[di_v.at[j]], sem).wait()

    plsc.subcore_barrier()
    pltpu.sync_copy(acc_sh.at[pl.ds(s * ROWS_PT, ROWS_PT)],
                    out_hbm.at[c, pl.ds(s * ROWS_PT, ROWS_PT)])

  return deg


@functools.lru_cache(maxsize=None)
def _make_agg(n_chunks):
  cpt = n_chunks // NW + (1 if n_chunks % NW else 0)
  ng = -(-cpt // NBUF)

  @functools.partial(
      pl.kernel,
      out_type=jax.ShapeDtypeStruct((NC, NPAD, F), jnp.float32),
      mesh=_mesh,
      compiler_params=pltpu.CompilerParams(use_tc_tiling_on_sc=False),
      scratch_types=[
          pltpu.VMEM((cpt, CHUNK), jnp.int32),
          pltpu.VMEM((cpt, CHUNK), jnp.int32),
          pltpu.VMEM((NBUF, CHUNK, F), jnp.float32),
          pltpu.VMEM_SHARED((NPAD, F), jnp.float32),
          pltpu.VMEM_SHARED((NPAD, F), jnp.float32),
          pltpu.SemaphoreType.DMA,
          pltpu.SemaphoreType.DMA,
          pltpu.SemaphoreType.DMA((NBUF,)),
          pltpu.SemaphoreType.DMA((NBUF,)),
      ],
  )
  def agg(g_hbm, ei_hbm, zrows_hbm, out_hbm,
          si_v, di_v, rows_v, acc_sh, g_sh, sem1, sem2, gsem, ssem):
    c = lax.axis_index("c")
    s = lax.axis_index("s")
    base, cnt = _worker_span(c, s, n_chunks)
    per = n_chunks // NW
    cp1 = pltpu.async_copy(ei_hbm.at[0, pl.ds(base, per)],
                           si_v.at[pl.ds(0, per)], sem1)
    cp2 = pltpu.async_copy(ei_hbm.at[1, pl.ds(base, per)],
                           di_v.at[pl.ds(0, per)], sem2)
    pltpu.sync_copy(g_hbm.at[pl.ds(s * ROWS_PT, ROWS_PT)],
                    g_sh.at[pl.ds(s * ROWS_PT, ROWS_PT)])
    pltpu.sync_copy(zrows_hbm, acc_sh.at[pl.ds(s * ROWS_PT, ROWS_PT)])
    cp1.wait()
    cp2.wait()

    @pl.when(cnt > per)
    def _():
      pltpu.sync_copy(ei_hbm.at[0, pl.ds(base + per, 1)],
                      si_v.at[pl.ds(per, 1)])
      pltpu.sync_copy(ei_hbm.at[1, pl.ds(base + per, 1)],
                      di_v.at[pl.ds(per, 1)])

    plsc.subcore_barrier()

    def g_start(j, b):
      pltpu.async_copy(g_sh.at[si_v.at[j]], rows_v.at[b], gsem.at[b])

    def g_wait(j, b):
      pltpu.make_async_copy(g_sh.at[si_v.at[j]], rows_v.at[b],
                            gsem.at[b]).wait()

    def s_start(j, b):
      pltpu.async_copy(rows_v.at[b], acc_sh.at[di_v.at[j]], ssem.at[b],
                       add=True)

    def s_wait(j, b):
      pltpu.make_async_copy(rows_v.at[b], acc_sh.at[di_v.at[j]],
                            ssem.at[b]).wait()

    for b in range(NBUF):
      g_start(b, b)

    def outer(g, carry):
      gbase = g * NBUF
      for b in range(NBUF):
        j = gbase + b

        @pl.when(j < cnt)
        def _():
          g_wait(j, b)
          s_start(j, b)

      for b in range(NBUF):
        j = gbase + b
        j2 = j + NBUF

        @pl.when(j < cnt)
        def _():
          s_wait(j, b)

        @pl.when(j2 < cnt)
        def _():
          g_start(j2, b)

      return carry

    lax.fori_loop(0, ng, outer, 0)
    plsc.subcore_barrier()
    pltpu.sync_copy(acc_sh.at[pl.ds(s * ROWS_PT, ROWS_PT)],
                    out_hbm.at[c, pl.ds(s * ROWS_PT, ROWS_PT)])

  return agg


def _tc1a(x3, W1):
  def body(x_ref, w1_ref, h1_ref):
    for a in range(8):
      h1_ref[:PKN, 16 * a:16 * (a + 1)] = jnp.dot(
          x_ref[:, a, :], w1_ref[...], preferred_element_type=jnp.float32)
    h1_ref[PKN:, :] = jnp.zeros((PK - PKN, 128), jnp.float32)

  return pl.pallas_call(
      body,
      out_shape=jax.ShapeDtypeStruct((PK, 128), jnp.float32),
  )(x3, W1)


def _tc1b(degp, h1):
  def body(degp_ref, h1_ref, g1_ref, dinv_ref):
    dinv = lax.rsqrt(degp_ref[0] + degp_ref[1] + 1.0)
    dinv_ref[...] = dinv
    g1_ref[...] = h1_ref[...] * dinv

  return pl.pallas_call(
      body,
      out_shape=(jax.ShapeDtypeStruct((PK, 128), jnp.float32),
                 jax.ShapeDtypeStruct((PK, 128), jnp.float32)),
  )(degp, h1)


def _tc2(accp, g1, dinv, b1t, W2):
  def body(accp_ref, g1_ref, dinv_ref, b1t_ref, w2_ref, g2_ref):
    acc = accp_ref[0] + accp_ref[1] + g1_ref[...]
    h = jnp.maximum(dinv_ref[...] * acc + b1t_ref[...], 0.0)
    for a in range(8):
      ha = h[:, 16 * a:16 * (a + 1)]
      g2a = jnp.dot(ha, w2_ref[...], preferred_element_type=jnp.float32)
      g2_ref[:, 16 * a:16 * (a + 1)] = g2a * dinv_ref[:, 16 * a:16 * (a + 1)]

  return pl.pallas_call(
      body,
      out_shape=jax.ShapeDtypeStruct((PK, 128), jnp.float32),
  )(accp, g1, dinv, b1t, W2)


def _tc3(accp, g2, dinv, b2t, Wf1, bf1, Wf2, bf2):
  def body(accp_ref, g2_ref, dinv_ref, b2t_ref, wf1_ref, bf1_ref,
           wf2_ref, bf2_ref, out_ref):
    acc = accp_ref[0] + accp_ref[1] + g2_ref[...]
    h = jnp.maximum(dinv_ref[...] * acc + b2t_ref[...], 0.0)
    cols = []
    for a in range(8):
      ha = h[:, 16 * a:16 * (a + 1)]
      m = jnp.maximum(
          jnp.dot(ha, wf1_ref[...], preferred_element_type=jnp.float32)
          + bf1_ref[...], 0.0)
      cols.append(jnp.dot(m, wf2_ref[...], preferred_element_type=jnp.float32)
                  + bf2_ref[...])
    out_ref[...] = jnp.concatenate(cols, axis=1)

  return pl.pallas_call(
      body,
      out_shape=jax.ShapeDtypeStruct((PK, 8), jnp.float32),
  )(accp, g2, dinv, b2t, Wf1, bf1, Wf2, bf2)


def kernel(x, edge_index, W1, b1, W2, b2, Wf1, bf1, Wf2, bf2):
  E = edge_index.shape[1]
  n_chunks = E // CHUNK
  ei3 = edge_index.astype(jnp.int32).reshape(2, n_chunks, CHUNK)
  x3 = x.reshape(PKN, 8, 128)
  zrows = jnp.zeros((ROWS_PT, F), jnp.float32)
  ones = jnp.ones((CHUNK, F), jnp.float32)
  b1t = jnp.tile(b1, 8).reshape(1, 128)
  b2t = jnp.tile(b2, 8).reshape(1, 128)

  degp = _make_deg(n_chunks)(ei3, zrows, ones)
  h1 = _tc1a(x3, W1)
  degp_pk = degp.reshape(NC, PK, 128)
  g1, dinv = _tc1b(degp_pk, h1)
  acc1 = _make_agg(n_chunks)(g1.reshape(NPAD, F), ei3, zrows)
  g2 = _tc2(acc1.reshape(NC, PK, 128), g1, dinv, b1t, W2)
  acc2 = _make_agg(n_chunks)(g2.reshape(NPAD, F), ei3, zrows)
  out_pk = _tc3(acc2.reshape(NC, PK, 128), g2, dinv, b2t, Wf1,
                bf1.reshape(1, 8), Wf2, bf2.reshape(1, 1))
  return out_pk.reshape(NPAD, 1)[:N]
